# baseline (speedup 1.0000x reference)
import functools

import jax
import jax.numpy as jnp
from jax import lax
from jax.experimental import pallas as pl
from jax.experimental.pallas import tpu as pltpu
from jax.experimental.pallas import tpu_sc as plsc

KNN = 16
N_PTS = 2048
BS = 8
NBIG = 2048


def _bdot(a16, b16):
    return lax.dot_general(a16, b16, (((1,), (0,)), ((), ())),
                           preferred_element_type=jnp.float32)


def _topk_body(with_ac, xr_ref, xa_t_ref, *rest):
    if with_ac:
        wa_ref, wd_ref, b_ref, idx_ref, a_ref, c_ref = rest
    else:
        idx_ref, = rest
    xr = xr_ref[0]
    xat = xa_t_ref[0]
    R = xr.shape[0]
    inner = -2.0 * _bdot(xr.astype(jnp.bfloat16), xat.astype(jnp.bfloat16))
    sq_r = jnp.sum(xr * xr, axis=1, keepdims=True)
    sq_a = jnp.sum(xat * xat, axis=0, keepdims=True)
    neg = -((sq_r + inner) + sq_a)
    iota_f = lax.broadcasted_iota(jnp.int32, neg.shape, 1).astype(jnp.float32)
    off = pl.program_id(0) * N_PTS
    cols = []
    for _ in range(KNN):
        m = jnp.max(neg, axis=1, keepdims=True)
        cand = jnp.where(neg == m, iota_f, jnp.float32(NBIG))
        j = jnp.min(cand, axis=1, keepdims=True)
        neg = jnp.where(iota_f == j, -jnp.inf, neg)
        cols.append(j)
    idx_f = jnp.concatenate(cols, axis=1)
    idx_ref[0] = idx_f.astype(jnp.int32) + off
    if with_ac:
        hi = jax.lax.Precision.HIGHEST
        a_ref[...] = lax.dot_general(xr, wa_ref[...], (((1,), (0,)), ((), ())),
                                     precision=hi)
        c_ref[...] = lax.dot_general(xr, wd_ref[...], (((1,), (0,)), ((), ())),
                                     precision=hi) + b_ref[...]


def _topk(x, xt, wa=None, wd=None, bias=None, row_tile=256, out_dim=None):
    bs, n, c = x.shape
    t = n // row_tile
    grid = (bs, t)
    in_specs = [
        pl.BlockSpec((1, row_tile, c), lambda bi, ti: (bi, ti, 0)),
        pl.BlockSpec((1, c, n), lambda bi, ti: (bi, 0, 0)),
    ]
    out_shapes = [jax.ShapeDtypeStruct((bs, n, KNN), jnp.int32)]
    out_specs = [pl.BlockSpec((1, row_tile, KNN), lambda bi, ti: (bi, ti, 0))]
    with_ac = wa is not None
    if with_ac:
        o = out_dim
        in_specs += [
            pl.BlockSpec((c, o), lambda bi, ti: (0, 0)),
            pl.BlockSpec((c, o), lambda bi, ti: (0, 0)),
            pl.BlockSpec((1, o), lambda bi, ti: (0, 0)),
        ]
        out_shapes += [jax.ShapeDtypeStruct((bs * n, o), jnp.float32),
                       jax.ShapeDtypeStruct((bs * n, o), jnp.float32)]
        out_specs += [pl.BlockSpec((row_tile, o), lambda bi, ti: (bi * t + ti, 0)),
                      pl.BlockSpec((row_tile, o), lambda bi, ti: (bi * t + ti, 0))]
        args = (x, xt, wa, wd, bias)
    else:
        args = (x, xt)
    return pl.pallas_call(
        functools.partial(_topk_body, with_ac),
        grid=grid,
        in_specs=in_specs,
        out_specs=out_specs,
        out_shape=out_shapes,
        compiler_params=pltpu.CompilerParams(
            dimension_semantics=("parallel", "parallel")),
    )(*args)


def _sc_gather(table, idx, chunk=256, out_width=None):
    v, d = table.shape
    ow = d if out_width is None else out_width
    e = idx.shape[0]
    nw = 32
    b_per_w = e // nw
    nchunks = b_per_w // chunk
    mesh = plsc.VectorSubcoreMesh(core_axis_name="c", subcore_axis_name="s")

    pack = 1 if ow == d else 128 // ow
    out_rows = e // pack
    out_cols = d if ow == d else 128
    assert nchunks % 2 == 0
    v2 = v // 2
    scratch = [pltpu.VMEM((chunk,), jnp.int32),
               pltpu.VMEM((chunk,), jnp.int32),
               pltpu.VMEM((2, chunk, d), jnp.float32),
               pltpu.VMEM_SHARED((v2, d), jnp.float32)]
    if pack > 1:
        scratch.append(pltpu.VMEM((2, chunk // pack, 128), jnp.float32))
    scratch += [pltpu.SemaphoreType.DMA] * 4

    @functools.partial(
        pl.kernel,
        out_type=jax.ShapeDtypeStruct((out_rows, out_cols), jnp.float32),
        mesh=mesh,
        scratch_types=scratch,
    )
    def k(table_hbm, idx_hbm, out_hbm, idx_v0, idx_v1, rows_v, table_sh, *rest):
        if pack > 1:
            slim_v = rest[0]
            rest = rest[1:]
        semg0, semg1, semw0, semw1 = rest
        sid = lax.axis_index("s")
        cid = lax.axis_index("c")
        base = cid * (e // 2) + sid * b_per_w
        tbase = cid * v2

        @pl.when(sid == 0)
        def _():
            pltpu.sync_copy(table_hbm.at[pl.ds(tbase, v2)], table_sh)
        plsc.subcore_barrier()

        def localize(idx_v):
            @pl.loop(0, chunk, step=16)
            def _(t):
                sl = pl.ds(t, 16)
                idx_v[sl] = idx_v[sl] - tbase

        def emit_writeback(buf, b0, sem):
            if pack == 1:
                return pltpu.async_copy(rows_v.at[buf],
                                        out_hbm.at[pl.ds(b0, chunk)], sem)

            @pl.loop(0, chunk // pack)
            def _(p):
                for q in range(pack):
                    slim_v[buf, p, pl.ds(q * ow, ow)] = (
                        rows_v[buf, p * pack + q, pl.ds(0, ow)])
            o0 = pl.multiple_of(b0 // pack, chunk // pack)
            return pltpu.async_copy(slim_v.at[buf],
                                    out_hbm.at[pl.ds(o0, chunk // pack)], sem)

        @pl.loop(0, nchunks, step=2)
        def _(i):
            b0 = base + i * chunk
            b1 = b0 + chunk
            pltpu.sync_copy(idx_hbm.at[pl.ds(b0, chunk)], idx_v0)
            localize(idx_v0)
            ga = pltpu.async_copy(table_sh.at[idx_v0], rows_v.at[0], semg0)
            pltpu.sync_copy(idx_hbm.at[pl.ds(b1, chunk)], idx_v1)
            localize(idx_v1)
            ga.wait()
            gb = pltpu.async_copy(table_sh.at[idx_v1], rows_v.at[1], semg1)
            wa = emit_writeback(0, b0, semw0)
            gb.wait()
            wb = emit_writeback(1, b1, semw1)
            wa.wait()
            wb.wait()

    return k(table, idx)


def _sc_gather_max(table, idx, cep, chunk=256):
    v, d = table.shape
    e = idx.shape[0]
    nw = 32
    b_per_w = e // nw
    nchunks = b_per_w // chunk
    assert nchunks % 2 == 0 and chunk % KNN == 0
    pts = chunk // KNN
    mesh = plsc.VectorSubcoreMesh(core_axis_name="c", subcore_axis_name="s")

    v2 = v // 2

    @functools.partial(
        pl.kernel,
        out_type=jax.ShapeDtypeStruct((v, d), jnp.float32),
        mesh=mesh,
        scratch_types=[pltpu.VMEM((chunk,), jnp.int32),
                       pltpu.VMEM((chunk,), jnp.int32),
                       pltpu.VMEM((2, chunk, d), jnp.float32),
                       pltpu.VMEM((2, pts, d), jnp.float32),
                       pltpu.VMEM((2, pts, d), jnp.float32),
                       pltpu.VMEM_SHARED((v2, d), jnp.float32)]
                      + [pltpu.SemaphoreType.DMA] * 6,
    )
    def k(table_hbm, idx_hbm, cep_hbm, out_hbm, idx_v0, idx_v1, rows_v,
          c_v, out_v, table_sh, semg0, semg1, semc0, semc1, semw0, semw1):
        sid = lax.axis_index("s")
        cid = lax.axis_index("c")
        base = cid * (e // 2) + sid * b_per_w
        tbase = cid * v2

        @pl.when(sid == 0)
        def _():
            pltpu.sync_copy(table_hbm.at[pl.ds(tbase, v2)], table_sh)
        plsc.subcore_barrier()

        def localize(idx_v):
            @pl.loop(0, chunk, step=16)
            def _(t):
                sl = pl.ds(t, 16)
                idx_v[sl] = idx_v[sl] - tbase

        def compute(buf, p0, semw):
            @pl.loop(0, pts)
            def _(p):
                for f in range(d // 16):
                    sl = pl.ds(f * 16, 16)
                    acc = rows_v[buf, p * KNN, sl]
                    for t in range(1, KNN):
                        acc = jnp.maximum(acc, rows_v[buf, p * KNN + t, sl])
                    h = acc + c_v[buf, p, sl]
                    out_v[buf, p, sl] = jnp.maximum(h, 0.2 * h)
            return pltpu.async_copy(out_v.at[buf], out_hbm.at[pl.ds(p0, pts)],
                                    semw)

        @pl.loop(0, nchunks, step=2)
        def _(i):
            b0 = base + i * chunk
            b1 = b0 + chunk
            p0 = pl.multiple_of(b0 // KNN, pts)
            p1 = pl.multiple_of(b1 // KNN, pts)
            pltpu.sync_copy(idx_hbm.at[pl.ds(b0, chunk)], idx_v0)
            localize(idx_v0)
            ga = pltpu.async_copy(table_sh.at[idx_v0], rows_v.at[0], semg0)
            ca = pltpu.async_copy(cep_hbm.at[pl.ds(p0, pts)], c_v.at[0], semc0)
            pltpu.sync_copy(idx_hbm.at[pl.ds(b1, chunk)], idx_v1)
            localize(idx_v1)
            cb = pltpu.async_copy(cep_hbm.at[pl.ds(p1, pts)], c_v.at[1], semc1)
            ga.wait()
            gb = pltpu.async_copy(table_sh.at[idx_v1], rows_v.at[1], semg1)
            ca.wait()
            wa = compute(0, p0, semw0)
            gb.wait()
            cb.wait()
            wb = compute(1, p1, semw1)
            wa.wait()
            wb.wait()

    return k(table, idx, cep)


def _edge1_body(g_ref, ctr_ref, wa_ref, wb_ref, b_ref, out_ref):
    rn = ctr_ref.shape[0]
    cpad = ctr_ref.shape[1]
    g = g_ref[...]
    ctr = ctr_ref[...]
    diff = g - ctr[:, None, :]
    diff2 = diff.reshape(rn * KNN, cpad).astype(jnp.bfloat16)
    ctr2 = jnp.broadcast_to(ctr[:, None, :], (rn, KNN, cpad))
    ctr2 = ctr2.reshape(rn * KNN, cpad).astype(jnp.bfloat16)
    h = _bdot(diff2, wa_ref[...]) + _bdot(ctr2, wb_ref[...]) + b_ref[...]
    h = jnp.max(h.reshape(rn, KNN, h.shape[-1]), axis=1)
    out_ref[...] = jnp.where(h > 0, h, 0.2 * h)


def _edge1(g, xpad, wa16, wb16, bias, row_tile=256):
    n, _, cpad = g.shape
    o = wa16.shape[1]
    grid = (n // row_tile,)
    return pl.pallas_call(
        _edge1_body,
        grid=grid,
        in_specs=[
            pl.BlockSpec((row_tile, KNN, cpad), lambda i: (i, 0, 0)),
            pl.BlockSpec((row_tile, cpad), lambda i: (i, 0)),
            pl.BlockSpec((cpad, o), lambda i: (0, 0)),
            pl.BlockSpec((cpad, o), lambda i: (0, 0)),
            pl.BlockSpec((1, o), lambda i: (0, 0)),
        ],
        out_specs=pl.BlockSpec((row_tile, o), lambda i: (i, 0)),
        out_shape=jax.ShapeDtypeStruct((n, o), jnp.float32),
        compiler_params=pltpu.CompilerParams(dimension_semantics=("parallel",)),
    )(g, xpad, wa16, wb16, bias)


STREAMS = 2


def kernel(xy, W1, b1, W2, b2):
    bs, n_stk, n_stk_pnt, pd = xy.shape
    n = n_stk * n_stk_pnt
    xb = xy.reshape(bs, n, pd)

    cpad = 128
    slim = 16
    xpad = jnp.pad(xb, ((0, 0), (0, 0), (0, cpad - pd)))
    xpad_t = jnp.transpose(xpad, (0, 2, 1))
    w1a = jnp.zeros((slim, 64), jnp.float32).at[:pd].set(W1[:, :pd].T)
    w1b = jnp.zeros((slim, 64), jnp.float32).at[:pd].set(W1[:, pd:].T)
    w1a16, w1b16 = w1a.astype(jnp.bfloat16), w1b.astype(jnp.bfloat16)
    w2a = W2[:, :64].T
    w2d = (W2[:, 64:] - W2[:, :64]).T
    b1r, b2r = b1.reshape(1, 64), b2.reshape(1, 128)

    bs_s = bs // STREAMS
    ng = bs_s * n
    xps, xpts, idx1s, g1s, h1s, l2s, outs = [], [], [], [], [], [], []
    for s in range(STREAMS):
        xps.append(lax.slice_in_dim(xpad, s * bs_s, (s + 1) * bs_s, axis=0))
        xpts.append(lax.slice_in_dim(xpad_t, s * bs_s, (s + 1) * bs_s, axis=0))
    for s in range(STREAMS):
        idx1s.append(_topk(xps[s], xpts[s])[0])
    for s in range(STREAMS):
        g1s.append(_sc_gather(xps[s].reshape(ng, cpad),
                              idx1s[s].reshape(ng * KNN),
                              out_width=slim))
    for s in range(STREAMS):
        h1s.append(_edge1(g1s[s].reshape(ng, KNN, slim),
                          xps[s].reshape(ng, cpad)[:, :slim],
                          w1a16, w1b16, b1r))
    for s in range(STREAMS):
        h1b = h1s[s].reshape(bs_s, n, 64)
        h1t = jnp.transpose(h1b, (0, 2, 1))
        l2s.append(_topk(h1b, h1t, w2a, w2d, b2r, out_dim=128))
    for s in range(STREAMS):
        idx2, a2, c2 = l2s[s]
        outs.append(_sc_gather_max(a2, idx2.reshape(ng * KNN), c2))

    out = jnp.concatenate(outs, axis=0)
    out = out.reshape(bs, n, 128).transpose(0, 2, 1)
    return out.reshape(bs, 128, n_stk, n_stk_pnt)
